# baseline (speedup 1.0000x reference)
import jax
import jax.numpy as jnp
from jax.experimental import pallas as pl

N_ROIS = 5000
IN_CH = 256
ROI = 7
N_OUT = 32 + 124 + 3 + 7 + 2
M_BLK = 200


def _fused_heads_kernel(x_ref, w_ref, b_ref, o_ref):
    acc = b_ref[...].astype(jnp.float32)
    for i in range(ROI):
        for j in range(ROI):
            xs = x_ref[i, j].astype(jnp.bfloat16)
            acc = acc + jnp.dot(xs, w_ref[j, i],
                                preferred_element_type=jnp.float32)
    o_ref[...] = acc


def _fused_matmul(xt, w4, b_all):
    grid = (N_ROIS // M_BLK,)
    return pl.pallas_call(
        _fused_heads_kernel,
        grid=grid,
        in_specs=[
            pl.BlockSpec((ROI, ROI, M_BLK, IN_CH), lambda i: (0, 0, i, 0)),
            pl.BlockSpec((ROI, ROI, IN_CH, N_OUT), lambda i: (0, 0, 0, 0)),
            pl.BlockSpec((1, N_OUT), lambda i: (0, 0)),
        ],
        out_specs=pl.BlockSpec((M_BLK, N_OUT), lambda i: (i, 0)),
        out_shape=jax.ShapeDtypeStruct((N_ROIS, N_OUT), jnp.float32),
    )(xt, w4, b_all)


def kernel(x, W_cls, b_cls, W_reg, b_reg, W_face, b_face, W_colour, b_colour, W_motion, b_motion):
    xt = jnp.transpose(x, (2, 3, 0, 1))
    w_all = jnp.concatenate([W_cls, W_reg, W_face, W_colour, W_motion], axis=0)
    w4 = w_all.astype(jnp.bfloat16).reshape(N_OUT, IN_CH, ROI, ROI).T
    b_all = jnp.concatenate([b_cls, b_reg, b_face, b_colour, b_motion])[None, :]
    out = _fused_matmul(xt, w4, b_all)
    n_cls = W_cls.shape[0]
    n_reg = W_reg.shape[0]
    n_face = W_face.shape[0]
    n_colour = W_colour.shape[0]
    o1 = n_cls
    o2 = o1 + n_reg
    o3 = o2 + n_face
    o4 = o3 + n_colour
    return (
        out[:, :o1],
        out[:, o1:o2],
        out[:, o2:o3],
        out[:, o3:o4],
        out[:, o4:],
    )


# speedup vs baseline: 1.3765x; 1.3765x over previous
import jax
import jax.numpy as jnp
from jax.experimental import pallas as pl
from jax.experimental.pallas import tpu as pltpu

N_ROIS = 5000
IN_CH = 256
ROI = 7
N_SPATIAL = ROI * ROI
N_PAD = 184
M_BLK = 128
GRID_M = (N_ROIS + M_BLK - 1) // M_BLK

OFF_CLS, OFF_REG, OFF_FACE, OFF_COLOUR, OFF_MOTION = 0, 32, 160, 168, 176
SMALL0 = 160


def _fused_heads_kernel(x_ref, wc_ref, wr_ref, wf_ref, wl_ref, wm_ref,
                        b_ref, oc_ref, or_ref, os_ref, wscr):
    @pl.when(pl.program_id(0) == 0)
    def _build_weight_slabs():
        heads = ((wc_ref, OFF_CLS, 32), (wr_ref, OFF_REG, 124),
                 (wf_ref, OFF_FACE, 3), (wl_ref, OFF_COLOUR, 7),
                 (wm_ref, OFF_MOTION, 2))
        for s in range(N_SPATIAL):
            for wref, off, n in heads:
                wscr[s, :, off:off + n] = wref[s::N_SPATIAL, :].astype(jnp.bfloat16)

    acc = b_ref[...].astype(jnp.float32)
    for i in range(ROI):
        for j in range(ROI):
            xs = x_ref[i, j].astype(jnp.bfloat16)
            acc = acc + jnp.dot(xs, wscr[i * ROI + j],
                                preferred_element_type=jnp.float32)
    or_ref[...] = acc[:, OFF_REG:OFF_REG + 124]
    oc_ref[...] = acc[:, OFF_CLS:OFF_CLS + 32].T
    os_ref[...] = acc[:, SMALL0:N_PAD].T


def _fused_matmul(xt, wts, b_pad):
    return pl.pallas_call(
        _fused_heads_kernel,
        grid=(GRID_M,),
        in_specs=[
            pl.BlockSpec((ROI, ROI, M_BLK, IN_CH), lambda i: (0, 0, i, 0)),
            pl.BlockSpec((IN_CH * N_SPATIAL, 32), lambda i: (0, 0)),
            pl.BlockSpec((IN_CH * N_SPATIAL, 124), lambda i: (0, 0)),
            pl.BlockSpec((IN_CH * N_SPATIAL, 3), lambda i: (0, 0)),
            pl.BlockSpec((IN_CH * N_SPATIAL, 7), lambda i: (0, 0)),
            pl.BlockSpec((IN_CH * N_SPATIAL, 2), lambda i: (0, 0)),
            pl.BlockSpec((1, N_PAD), lambda i: (0, 0)),
        ],
        out_specs=[
            pl.BlockSpec((32, M_BLK), lambda i: (0, i)),
            pl.BlockSpec((M_BLK, 124), lambda i: (i, 0)),
            pl.BlockSpec((24, M_BLK), lambda i: (0, i)),
        ],
        out_shape=[
            jax.ShapeDtypeStruct((32, N_ROIS), jnp.float32),
            jax.ShapeDtypeStruct((N_ROIS, 124), jnp.float32),
            jax.ShapeDtypeStruct((24, N_ROIS), jnp.float32),
        ],
        scratch_shapes=[pltpu.VMEM((N_SPATIAL, IN_CH, N_PAD), jnp.bfloat16)],
    )(xt, *wts, b_pad)


def kernel(x, W_cls, b_cls, W_reg, b_reg, W_face, b_face, W_colour, b_colour, W_motion, b_motion):
    xt = jnp.transpose(x, (2, 3, 0, 1))
    wts = (W_cls.T, W_reg.T, W_face.T, W_colour.T, W_motion.T)
    z = jnp.zeros((6,), jnp.float32)
    b_pad = jnp.concatenate(
        [b_cls, b_reg, z[:4], b_face, z[:5], b_colour, z[:1], b_motion, z])[None, :]
    oc, orr, osm = _fused_matmul(xt, wts, b_pad)
    return (
        oc.T,
        orr,
        osm[0:3].T,
        osm[8:15].T,
        osm[16:18].T,
    )


# speedup vs baseline: 1.6690x; 1.2125x over previous
import jax
import jax.numpy as jnp
from jax.experimental import pallas as pl
from jax.experimental.pallas import tpu as pltpu

N_ROIS = 5000
IN_CH = 256
ROI = 7
N_SPATIAL = ROI * ROI
FEAT = IN_CH * N_SPATIAL
N_PAD = 184
M_BLK = 256
GRID_M = (N_ROIS + M_BLK - 1) // M_BLK

OFF_REG, OFF_CLS, OFF_FACE, OFF_COLOUR, OFF_MOTION = 0, 128, 160, 168, 176
SMALL0 = 160


def _fused_heads_kernel(x_ref, wta_ref, wtb_ref, b_ref, oc_ref, or_ref, os_ref, wscr):
    @pl.when(pl.program_id(0) == 0)
    def _build_weight_slabs():
        for s in range(N_SPATIAL):
            wscr[s, :, 0:128] = wta_ref[s::N_SPATIAL, :].astype(jnp.bfloat16)
            wscr[s, :, 128:N_PAD] = wtb_ref[s::N_SPATIAL, :].astype(jnp.bfloat16)

    acc = b_ref[...].astype(jnp.float32)
    for i in range(ROI):
        for j in range(ROI):
            xs = x_ref[i, j].astype(jnp.bfloat16)
            acc = acc + jnp.dot(xs, wscr[i * ROI + j],
                                preferred_element_type=jnp.float32)
    or_ref[...] = acc[:, OFF_REG:OFF_REG + 124]
    oc_ref[...] = acc[:, OFF_CLS:OFF_CLS + 32].T
    os_ref[...] = acc[:, SMALL0:N_PAD].T


def _fused_matmul(xt, wta, wtb, b_pad):
    return pl.pallas_call(
        _fused_heads_kernel,
        grid=(GRID_M,),
        in_specs=[
            pl.BlockSpec((ROI, ROI, M_BLK, IN_CH), lambda i: (0, 0, i, 0)),
            pl.BlockSpec((FEAT, 128), lambda i: (0, 0)),
            pl.BlockSpec((FEAT, 56), lambda i: (0, 0)),
            pl.BlockSpec((1, N_PAD), lambda i: (0, 0)),
        ],
        out_specs=[
            pl.BlockSpec((32, M_BLK), lambda i: (0, i)),
            pl.BlockSpec((M_BLK, 124), lambda i: (i, 0)),
            pl.BlockSpec((24, M_BLK), lambda i: (0, i)),
        ],
        out_shape=[
            jax.ShapeDtypeStruct((32, N_ROIS), jnp.float32),
            jax.ShapeDtypeStruct((N_ROIS, 124), jnp.float32),
            jax.ShapeDtypeStruct((24, N_ROIS), jnp.float32),
        ],
        scratch_shapes=[pltpu.VMEM((N_SPATIAL, IN_CH, N_PAD), jnp.bfloat16)],
    )(xt, wta, wtb, b_pad)


def kernel(x, W_cls, b_cls, W_reg, b_reg, W_face, b_face, W_colour, b_colour, W_motion, b_motion):
    xt = jnp.transpose(x, (2, 3, 0, 1))
    zw = jnp.zeros((6, FEAT), jnp.float32)
    wta = jnp.concatenate([W_reg, zw[:4]], axis=0).T
    wtb = jnp.concatenate(
        [W_cls, W_face, zw[:5], W_colour, zw[:1], W_motion, zw], axis=0).T
    zb = jnp.zeros((6,), jnp.float32)
    b_pad = jnp.concatenate(
        [b_reg, zb[:4], b_cls, b_face, zb[:5], b_colour, zb[:1], b_motion, zb])[None, :]
    oc, orr, osm = _fused_matmul(xt, wta, wtb, b_pad)
    return (
        oc.T,
        orr,
        osm[0:3].T,
        osm[8:15].T,
        osm[16:18].T,
    )
